# baseline (speedup 1.0000x reference)
import functools

import jax
import jax.numpy as jnp
import numpy as np
from jax.experimental import pallas as pl

_T = 512
_DH = 64
_R = 8
_U = _T // _R


def _head_kernel(q_ref, k_ref, v_ref, o_ref):
    q = q_ref[0, 0, 0]
    k = k_ref[0, 0, 0]
    v = v_ref[0, 0, 0]
    scale = np.float32(1.0 / np.sqrt(_DH))

    q3 = q.reshape(_U, _R, _DH)
    k3 = k.reshape(_U, _R, _DH)
    v3 = v.reshape(_U, _R, _DH)
    s = jax.lax.dot_general(
        q3, k3, dimension_numbers=(((2,), (2,)), ((1,), (1,))),
        precision=jax.lax.Precision.HIGHEST,
        preferred_element_type=jnp.float32)
    e = jnp.exp(scale * s) - 1.0
    v3a = jnp.concatenate([v3, jnp.ones((_U, _R, 1), jnp.float32)], axis=2)
    num_s = jax.lax.dot_general(
        e, v3a, dimension_numbers=(((2,), (0,)), ((0,), (1,))),
        precision=jax.lax.Precision.HIGHEST,
        preferred_element_type=jnp.float32)
    num_a = num_s.transpose(1, 0, 2).reshape(_T, _DH + 1)
    num = num_a[:, :_DH]
    den = num_a[:, _DH:]

    for d in (1, 2):
        wf = jnp.exp(scale * jnp.sum(q[: _T - d] * k[d:], axis=1,
                                     keepdims=True)) - 1.0
        num += jnp.pad(wf * v[d:], ((0, d), (0, 0)))
        den += jnp.pad(wf, ((0, d), (0, 0)))
        wb = jnp.exp(scale * jnp.sum(q[d:] * k[: _T - d], axis=1,
                                     keepdims=True)) - 1.0
        num += jnp.pad(wb * v[: _T - d], ((d, 0), (0, 0)))
        den += jnp.pad(wb, ((d, 0), (0, 0)))

    vsum = jnp.sum(v, axis=0, keepdims=True)
    o_ref[0, 0, 0] = (num + vsum) / (den + np.float32(_T))


@jax.jit
def _run(q, k, v):
    B, N, H, T, Dh = q.shape
    qk_spec = pl.BlockSpec((1, 1, 1, T, Dh), lambda b, n, h: (b, h, n, 0, 0))
    v_spec = pl.BlockSpec((1, 1, 1, T, Dh), lambda b, n, h: (b, n, h, 0, 0))
    out = pl.pallas_call(
        _head_kernel,
        grid=(B, N, H),
        in_specs=[qk_spec, qk_spec, v_spec],
        out_specs=v_spec,
        out_shape=jax.ShapeDtypeStruct((B, N, H, T, Dh), jnp.float32),
    )(q, k, v)
    out = out.transpose(0, 1, 3, 2, 4).reshape(B, N, T, H * Dh)
    return out.transpose(0, 2, 1, 3)


def kernel(q, k, v, dims):
    return _run(q, k, v)


# speedup vs baseline: 1.2957x; 1.2957x over previous
import jax
import jax.numpy as jnp
import numpy as np
from jax.experimental import pallas as pl

_T = 512
_DH = 64
_R = 8
_U = _T // _R
_HI = jax.lax.Precision.HIGHEST


def _shift_u_fwd(x):
    return jnp.concatenate(
        [x[:, 1:, :], jnp.zeros((x.shape[0], 1, _DH), jnp.float32)], axis=1)


def _shift_u_bwd(x):
    return jnp.concatenate(
        [jnp.zeros((x.shape[0], 1, _DH), jnp.float32), x[:, :-1, :]], axis=1)


def _head_kernel(q_ref, k_ref, v_ref, o_ref):
    q3 = q_ref[0, 0, 0]
    k3 = k_ref[0, 0, 0]
    v3 = v_ref[0, 0, 0]
    scale = np.float32(1.0 / np.sqrt(_DH))

    s = jax.lax.dot_general(
        q3, k3, dimension_numbers=(((2,), (2,)), ((0,), (0,))),
        precision=_HI, preferred_element_type=jnp.float32)
    e = jnp.exp(scale * s) - 1.0
    v3a = jnp.concatenate([v3, jnp.ones((_R, _U, 1), jnp.float32)], axis=2)
    num_a = jax.lax.dot_general(
        e, v3a, dimension_numbers=(((2,), (1,)), ((0,), (0,))),
        precision=_HI, preferred_element_type=jnp.float32)
    num = num_a[:, :, :_DH]
    den = num_a[:, :, _DH:]

    for d in (1, 2):
        kp = jnp.concatenate([k3[d:], _shift_u_fwd(k3[:d])], axis=0)
        vp = jnp.concatenate([v3[d:], _shift_u_fwd(v3[:d])], axis=0)
        wf = jnp.exp(scale * jnp.sum(q3 * kp, axis=2, keepdims=True)) - 1.0
        num += wf * vp
        den += wf
        km = jnp.concatenate([_shift_u_bwd(k3[_R - d:]), k3[:_R - d]], axis=0)
        vm = jnp.concatenate([_shift_u_bwd(v3[_R - d:]), v3[:_R - d]], axis=0)
        wb = jnp.exp(scale * jnp.sum(q3 * km, axis=2, keepdims=True)) - 1.0
        num += wb * vm
        den += wb

    vsum = jnp.sum(v3, axis=(0, 1), keepdims=True)
    o_ref[0, 0, 0] = (num + vsum) / (den + np.float32(_T))


@jax.jit
def _run(q, k, v):
    B, N, H, T, Dh = q.shape
    def to_res(x):
        return x.reshape(B, N, H, _U, _R, Dh).transpose(0, 1, 2, 4, 3, 5)
    qr, kr, vr = to_res(q), to_res(k), to_res(v)
    blk = (1, 1, 1, _R, _U, Dh)
    qk_spec = pl.BlockSpec(blk, lambda b, n, h: (b, h, n, 0, 0, 0))
    v_spec = pl.BlockSpec(blk, lambda b, n, h: (b, n, h, 0, 0, 0))
    out = pl.pallas_call(
        _head_kernel,
        grid=(B, N, H),
        in_specs=[qk_spec, qk_spec, v_spec],
        out_specs=v_spec,
        out_shape=jax.ShapeDtypeStruct((B, N, H, _R, _U, Dh), jnp.float32),
    )(qr, kr, vr)
    out = out.transpose(0, 1, 2, 4, 3, 5).reshape(B, N, H, T, Dh)
    out = out.transpose(0, 1, 3, 2, 4).reshape(B, N, T, H * Dh)
    return out.transpose(0, 2, 1, 3)


def kernel(q, k, v, dims):
    return _run(q, k, v)
